# baseline (speedup 1.0000x reference)
import functools

import jax
import jax.numpy as jnp
from jax import lax
from jax.experimental import pallas as pl
from jax.experimental.pallas import tpu as pltpu
from jax.experimental.pallas import tpu_sc as plsc

_B = 16384
_D = 64
_K = 4
_NC = 2
_NS = 16
_NW = _NC * _NS
_BW = _B // _NW
_C = 128
_NCHUNK = _BW // _C
_G = _C // 16
_PAIRS = ((0, 1), (0, 2), (0, 3), (1, 2), (1, 3), (2, 3))


def _rsqrt_clamped(x):
    x = jnp.maximum(x, jnp.float32(1e-24))
    xi = plsc.bitcast(x, jnp.int32)
    yi = jnp.int32(0x5F3759DF) - (xi >> 1)
    y = plsc.bitcast(yi, jnp.float32)
    for _ in range(3):
        y = y * (jnp.float32(1.5) - jnp.float32(0.5) * x * y * y)
    return y


def _sc_body(u_hbm, i_hbm, ue_hbm, ie_hbm, up_hbm,
             out_hbm, pair_hbm,
             u_v, i_v, uidx, iidx, urows, irows, prob_v,
             out_v, pair_v, sem):
    wid = lax.axis_index("s") * _NC + lax.axis_index("c")
    base = wid * _BW

    pltpu.sync_copy(u_hbm.at[pl.ds(base, _BW)], u_v)
    pltpu.sync_copy(i_hbm.at[pl.ds(base, _BW)], i_v)
    pltpu.sync_copy(up_hbm.at[pl.ds(wid * 16, 16)], prob_v)

    zero16 = jnp.zeros((16,), jnp.float32)
    for t in range(8):
        pair_v[t] = zero16

    lane = lax.iota(jnp.int32, 16)

    @pl.loop(0, _NCHUNK)
    def _chunk(c):
        cbase = c * _C
        for j in range(_C // 16):
            uv = u_v[pl.ds(cbase + j * 16, 16)]
            iv = i_v[pl.ds(cbase + j * 16, 16)]
            u2 = uv * 2
            i2 = iv * 2
            for h in range(2):
                uidx[h, pl.ds(j * 16, 16)] = u2 + h
                iidx[h, pl.ds(j * 16, 16)] = i2 + h

        cps = []
        for h in range(2):
            cps.append(pltpu.async_copy(
                ue_hbm.at[uidx.at[h]], urows.at[pl.ds(h * 128, 128)], sem))
            cps.append(pltpu.async_copy(
                ie_hbm.at[iidx.at[h]], irows.at[pl.ds(h * 128, 128)], sem))
        for cp in cps:
            cp.wait()

        @pl.loop(0, _G)
        def _group(g):
            rowb = lane + g * 16
            rowh = [rowb, rowb + 128]
            ui = [zero16] * _K
            uu = [zero16] * _K
            ii = [zero16] * _K
            pr = [zero16] * len(_PAIRS)
            stag = (lane * 5) & (_D - 1)
            for d in range(_D):
                dcol = (stag + d) & (_D - 1)
                cols = [dcol, dcol + 64]
                uvec = [plsc.load_gather(urows, [rowh[k >> 1], cols[k & 1]])
                        for k in range(_K)]
                ivec = [plsc.load_gather(irows, [rowh[k >> 1], cols[k & 1]])
                        for k in range(_K)]
                for k in range(_K):
                    ui[k] = ui[k] + uvec[k] * ivec[k]
                    uu[k] = uu[k] + uvec[k] * uvec[k]
                    ii[k] = ii[k] + ivec[k] * ivec[k]
                for t, (l, j) in enumerate(_PAIRS):
                    pr[t] = pr[t] + uvec[l] * uvec[j]

            rnu = [_rsqrt_clamped(uu[k]) for k in range(_K)]
            rni = [_rsqrt_clamped(ii[k]) for k in range(_K)]
            kdis = [ui[k] * rnu[k] * rni[k] for k in range(_K)]
            for t, (l, j) in enumerate(_PAIRS):
                pair_v[t] = pair_v[t] + pr[t] * rnu[l] * rnu[j]

            prow = jnp.full((16,), (c * _C + g * 16) >> 5, jnp.int32)
            pcol0 = lane * 4 + (g & 1) * 64
            p = [plsc.load_gather(prob_v, [prow, pcol0 + k])
                 for k in range(_K)]
            m = jnp.maximum(jnp.maximum(p[0], p[1]),
                            jnp.maximum(p[2], p[3]))
            e = [jnp.exp(p[k] - m) for k in range(_K)]
            num = e[0] * kdis[0] + e[1] * kdis[1] + e[2] * kdis[2] + e[3] * kdis[3]
            den = (e[0] + e[1]) + (e[2] + e[3])
            out_v[pl.ds(c * _C + g * 16, 16)] = num / den

    pltpu.sync_copy(out_v, out_hbm.at[pl.ds(base, _BW)])
    pltpu.sync_copy(pair_v, pair_hbm.at[wid])


_sc_call = pl.kernel(
    _sc_body,
    out_type=[
        jax.ShapeDtypeStruct((_B,), jnp.float32),
        jax.ShapeDtypeStruct((_NW, 8, 16), jnp.float32),
    ],
    mesh=plsc.VectorSubcoreMesh(core_axis_name="c", subcore_axis_name="s"),
    compiler_params=pltpu.CompilerParams(
        needs_layout_passes=False, use_tc_tiling_on_sc=True),
    scratch_types=[
        pltpu.VMEM((_BW,), jnp.int32),
        pltpu.VMEM((_BW,), jnp.int32),
        pltpu.VMEM((2, 128), jnp.int32),
        pltpu.VMEM((2, 128), jnp.int32),
        pltpu.VMEM((2 * _C, 128), jnp.float32),
        pltpu.VMEM((2 * _C, 128), jnp.float32),
        pltpu.VMEM((16, 128), jnp.float32),
        pltpu.VMEM((_BW,), jnp.float32),
        pltpu.VMEM((8, 16), jnp.float32),
        pltpu.SemaphoreType.DMA,
    ],
)


_RBLK = 640


def _tc_relayout_body(ueT_ref, ieT_ref, ue2_ref, ie2_ref):
    for src, dst in ((ueT_ref, ue2_ref), (ieT_ref, ie2_ref)):
        xt = src[...].T
        x3 = xt.reshape(_RBLK // 2, 2, 64)
        dst[...] = jnp.concatenate([x3[:, 0, :], x3[:, 1, :]], axis=1)


_tc_relayout = pl.pallas_call(
    _tc_relayout_body,
    grid=(400000 // _RBLK,),
    in_specs=[
        pl.BlockSpec((64, _RBLK), lambda j: (0, j)),
        pl.BlockSpec((64, _RBLK), lambda j: (0, j)),
    ],
    out_specs=[
        pl.BlockSpec((_RBLK // 2, 128), lambda j: (j, 0)),
        pl.BlockSpec((_RBLK // 2, 128), lambda j: (j, 0)),
    ],
    out_shape=[
        jax.ShapeDtypeStruct((200000, 128), jnp.float32),
        jax.ShapeDtypeStruct((200000, 128), jnp.float32),
    ],
)


def kernel(u, i, userEmbed_weight, itemEmbed_weight, userProb_weight):
    ue2, ie2 = _tc_relayout(userEmbed_weight.T, itemEmbed_weight.T)
    prob2 = jnp.take(userProb_weight, u, axis=0).reshape(_BW, 128)
    out, pair = _sc_call(u, i, ue2, ie2, prob2)
    s = jnp.sum(pair, axis=(0, 2))[:6]
    facet_loss = jnp.sum(jnp.log(1.0 + jnp.exp(-0.1 * s)))
    return out, facet_loss.astype(jnp.float32)


# speedup vs baseline: 1.9154x; 1.9154x over previous
import functools

import jax
import jax.numpy as jnp
from jax import lax
from jax.experimental import pallas as pl
from jax.experimental.pallas import tpu as pltpu
from jax.experimental.pallas import tpu_sc as plsc

_B = 16384
_D = 64
_K = 4
_NC = 2
_NS = 16
_NW = _NC * _NS
_BW = _B // _NW
_C = 128
_NCHUNK = _BW // _C
_G = _C // 16
_PAIRS = ((0, 1), (0, 2), (0, 3), (1, 2), (1, 3), (2, 3))


def _rsqrt_clamped(x):
    x = jnp.maximum(x, jnp.float32(1e-24))
    xi = plsc.bitcast(x, jnp.int32)
    yi = jnp.int32(0x5F3759DF) - (xi >> 1)
    y = plsc.bitcast(yi, jnp.float32)
    for _ in range(3):
        y = y * (jnp.float32(1.5) - jnp.float32(0.5) * x * y * y)
    return y


def _sc_body(u_hbm, i_hbm, ue_hbm, ie_hbm, up_hbm,
             out_hbm, pair_hbm,
             u_v, i_v, uidx, iidx, urows, irows, prob_v,
             out_v, pair_v, sem):
    wid = lax.axis_index("s") * _NC + lax.axis_index("c")
    base = wid * _BW

    pltpu.sync_copy(u_hbm.at[pl.ds(base, _BW)], u_v)
    pltpu.sync_copy(i_hbm.at[pl.ds(base, _BW)], i_v)
    pltpu.sync_copy(up_hbm.at[pl.ds(wid * 16, 16)], prob_v)

    zero16 = jnp.zeros((16,), jnp.float32)
    for t in range(8):
        pair_v[t] = zero16

    lane = lax.iota(jnp.int32, 16)

    @pl.loop(0, _NCHUNK)
    def _chunk(c):
        cbase = c * _C
        for j in range(_C // 16):
            uv = u_v[pl.ds(cbase + j * 16, 16)]
            iv = i_v[pl.ds(cbase + j * 16, 16)]
            u2 = uv * 2
            i2 = iv * 2
            for h in range(2):
                uidx[h, pl.ds(j * 16, 16)] = u2 + h
                iidx[h, pl.ds(j * 16, 16)] = i2 + h

        cps = []
        for h in range(2):
            cps.append(pltpu.async_copy(
                ue_hbm.at[uidx.at[h]], urows.at[pl.ds(h * 128, 128)], sem))
            cps.append(pltpu.async_copy(
                ie_hbm.at[iidx.at[h]], irows.at[pl.ds(h * 128, 128)], sem))
        for cp in cps:
            cp.wait()

        @pl.loop(0, _G)
        def _group(g):
            rowb = lane + g * 16
            rowh = [rowb, rowb + 128]
            ui = [zero16] * _K
            uu = [zero16] * _K
            ii = [zero16] * _K
            pr = [zero16] * len(_PAIRS)
            stag = (lane * 5) & (_D - 1)
            for d in range(_D):
                dcol = (stag + d) & (_D - 1)
                cols = [dcol, dcol + 64]
                uvec = [plsc.load_gather(urows, [rowh[k >> 1], cols[k & 1]])
                        for k in range(_K)]
                ivec = [plsc.load_gather(irows, [rowh[k >> 1], cols[k & 1]])
                        for k in range(_K)]
                for k in range(_K):
                    ui[k] = ui[k] + uvec[k] * ivec[k]
                    uu[k] = uu[k] + uvec[k] * uvec[k]
                    ii[k] = ii[k] + ivec[k] * ivec[k]
                for t, (l, j) in enumerate(_PAIRS):
                    pr[t] = pr[t] + uvec[l] * uvec[j]

            rnu = [_rsqrt_clamped(uu[k]) for k in range(_K)]
            rni = [_rsqrt_clamped(ii[k]) for k in range(_K)]
            kdis = [ui[k] * rnu[k] * rni[k] for k in range(_K)]
            for t, (l, j) in enumerate(_PAIRS):
                pair_v[t] = pair_v[t] + pr[t] * rnu[l] * rnu[j]

            prow = jnp.full((16,), (c * _C + g * 16) >> 5, jnp.int32)
            pcol0 = lane * 4 + (g & 1) * 64
            p = [plsc.load_gather(prob_v, [prow, pcol0 + k])
                 for k in range(_K)]
            m = jnp.maximum(jnp.maximum(p[0], p[1]),
                            jnp.maximum(p[2], p[3]))
            e = [jnp.exp(p[k] - m) for k in range(_K)]
            num = e[0] * kdis[0] + e[1] * kdis[1] + e[2] * kdis[2] + e[3] * kdis[3]
            den = (e[0] + e[1]) + (e[2] + e[3])
            out_v[pl.ds(c * _C + g * 16, 16)] = num / den

    pltpu.sync_copy(out_v, out_hbm.at[pl.ds(base, _BW)])
    pltpu.sync_copy(pair_v, pair_hbm.at[wid])


_sc_call = pl.kernel(
    _sc_body,
    out_type=[
        jax.ShapeDtypeStruct((_B,), jnp.float32),
        jax.ShapeDtypeStruct((_NW, 8, 16), jnp.float32),
    ],
    mesh=plsc.VectorSubcoreMesh(core_axis_name="c", subcore_axis_name="s"),
    compiler_params=pltpu.CompilerParams(
        needs_layout_passes=False, use_tc_tiling_on_sc=True),
    scratch_types=[
        pltpu.VMEM((_BW,), jnp.int32),
        pltpu.VMEM((_BW,), jnp.int32),
        pltpu.VMEM((2, 128), jnp.int32),
        pltpu.VMEM((2, 128), jnp.int32),
        pltpu.VMEM((2 * _C, 128), jnp.float32),
        pltpu.VMEM((2 * _C, 128), jnp.float32),
        pltpu.VMEM((16, 128), jnp.float32),
        pltpu.VMEM((_BW,), jnp.float32),
        pltpu.VMEM((8, 16), jnp.float32),
        pltpu.SemaphoreType.DMA,
    ],
)


_RBLK = 3200


def _tc_relayout_body(ueT_ref, ieT_ref, ue2_ref, ie2_ref):
    xcat = jnp.concatenate([ueT_ref[...], ieT_ref[...]], axis=0)
    xt = xcat.T
    for col0, dst in ((0, ue2_ref), (64, ie2_ref)):
        x3 = xt[:, col0:col0 + 64].reshape(_RBLK // 2, 2, 64)
        dst[...] = jnp.concatenate([x3[:, 0, :], x3[:, 1, :]], axis=1)


_tc_relayout = pl.pallas_call(
    _tc_relayout_body,
    grid=(400000 // _RBLK,),
    in_specs=[
        pl.BlockSpec((64, _RBLK), lambda j: (0, j)),
        pl.BlockSpec((64, _RBLK), lambda j: (0, j)),
    ],
    out_specs=[
        pl.BlockSpec((_RBLK // 2, 128), lambda j: (j, 0)),
        pl.BlockSpec((_RBLK // 2, 128), lambda j: (j, 0)),
    ],
    out_shape=[
        jax.ShapeDtypeStruct((200000, 128), jnp.float32),
        jax.ShapeDtypeStruct((200000, 128), jnp.float32),
    ],
)


def kernel(u, i, userEmbed_weight, itemEmbed_weight, userProb_weight):
    ue2, ie2 = _tc_relayout(userEmbed_weight.T, itemEmbed_weight.T)
    prob2 = jnp.take(userProb_weight, u, axis=0).reshape(_BW, 128)
    out, pair = _sc_call(u, i, ue2, ie2, prob2)
    s = jnp.sum(pair, axis=(0, 2))[:6]
    facet_loss = jnp.sum(jnp.log(1.0 + jnp.exp(-0.1 * s)))
    return out, facet_loss.astype(jnp.float32)


# speedup vs baseline: 2.1693x; 1.1325x over previous
import functools

import jax
import jax.numpy as jnp
from jax import lax
from jax.experimental import pallas as pl
from jax.experimental.pallas import tpu as pltpu
from jax.experimental.pallas import tpu_sc as plsc

_B = 16384
_D = 64
_K = 4
_NC = 2
_NS = 16
_NW = _NC * _NS
_BW = _B // _NW
_C = 128
_NCHUNK = _BW // _C
_G = _C // 16
_PAIRS = ((0, 1), (0, 2), (0, 3), (1, 2), (1, 3), (2, 3))


def _rsqrt_clamped(x):
    x = jnp.maximum(x, jnp.float32(1e-24))
    xi = plsc.bitcast(x, jnp.int32)
    yi = jnp.int32(0x5F3759DF) - (xi >> 1)
    y = plsc.bitcast(yi, jnp.float32)
    for _ in range(3):
        y = y * (jnp.float32(1.5) - jnp.float32(0.5) * x * y * y)
    return y


def _sc_body(u_hbm, i_hbm, ue_hbm, ie_hbm, up_hbm,
             out_hbm, pair_hbm,
             u_v, i_v, uidx, iidx, urows, irows, prob_v,
             out_v, pair_v, sem):
    wid = lax.axis_index("s") * _NC + lax.axis_index("c")
    base = wid * _BW

    pltpu.sync_copy(u_hbm.at[pl.ds(base, _BW)], u_v)
    pltpu.sync_copy(i_hbm.at[pl.ds(base, _BW)], i_v)
    pltpu.sync_copy(up_hbm.at[pl.ds(wid * 16, 16)], prob_v)

    zero16 = jnp.zeros((16,), jnp.float32)
    for t in range(8):
        pair_v[t] = zero16

    lane = lax.iota(jnp.int32, 16)

    @pl.loop(0, _NCHUNK)
    def _chunk(c):
        cbase = c * _C
        for j in range(_C // 16):
            uv = u_v[pl.ds(cbase + j * 16, 16)]
            iv = i_v[pl.ds(cbase + j * 16, 16)]
            u2 = uv * 2
            i2 = iv * 2
            for h in range(2):
                uidx[h, pl.ds(j * 16, 16)] = u2 + h
                iidx[h, pl.ds(j * 16, 16)] = i2 + h

        cps = []
        for h in range(2):
            cps.append(pltpu.async_copy(
                ue_hbm.at[uidx.at[h]], urows.at[pl.ds(h * 128, 128)], sem))
            cps.append(pltpu.async_copy(
                ie_hbm.at[iidx.at[h]], irows.at[pl.ds(h * 128, 128)], sem))
        for cp in cps:
            cp.wait()

        @pl.loop(0, _G)
        def _group(g):
            rowb = lane + g * 16
            rowh = [rowb, rowb + 128]
            ui = [zero16] * _K
            uu = [zero16] * _K
            ii = [zero16] * _K
            pr = [zero16] * len(_PAIRS)
            stag = (lane * 5) & (_D - 1)
            for d in range(_D):
                dcol = (stag + d) & (_D - 1)
                cols = [dcol, dcol + 64]
                uvec = [plsc.load_gather(urows, [rowh[k >> 1], cols[k & 1]])
                        for k in range(_K)]
                ivec = [plsc.load_gather(irows, [rowh[k >> 1], cols[k & 1]])
                        for k in range(_K)]
                for k in range(_K):
                    ui[k] = ui[k] + uvec[k] * ivec[k]
                    uu[k] = uu[k] + uvec[k] * uvec[k]
                    ii[k] = ii[k] + ivec[k] * ivec[k]
                for t, (l, j) in enumerate(_PAIRS):
                    pr[t] = pr[t] + uvec[l] * uvec[j]

            rnu = [_rsqrt_clamped(uu[k]) for k in range(_K)]
            rni = [_rsqrt_clamped(ii[k]) for k in range(_K)]
            kdis = [ui[k] * rnu[k] * rni[k] for k in range(_K)]
            for t, (l, j) in enumerate(_PAIRS):
                pair_v[t] = pair_v[t] + pr[t] * rnu[l] * rnu[j]

            prow = jnp.full((16,), (c * _C + g * 16) >> 5, jnp.int32)
            pcol0 = lane * 4 + (g & 1) * 64
            p = [plsc.load_gather(prob_v, [prow, pcol0 + k])
                 for k in range(_K)]
            m = jnp.maximum(jnp.maximum(p[0], p[1]),
                            jnp.maximum(p[2], p[3]))
            e = [jnp.exp(p[k] - m) for k in range(_K)]
            num = e[0] * kdis[0] + e[1] * kdis[1] + e[2] * kdis[2] + e[3] * kdis[3]
            den = (e[0] + e[1]) + (e[2] + e[3])
            out_v[pl.ds(c * _C + g * 16, 16)] = num / den

    pltpu.sync_copy(out_v, out_hbm.at[pl.ds(base, _BW)])
    pltpu.sync_copy(pair_v, pair_hbm.at[wid])


_sc_call = pl.kernel(
    _sc_body,
    out_type=[
        jax.ShapeDtypeStruct((_B,), jnp.float32),
        jax.ShapeDtypeStruct((_NW, 8, 16), jnp.float32),
    ],
    mesh=plsc.VectorSubcoreMesh(core_axis_name="c", subcore_axis_name="s"),
    compiler_params=pltpu.CompilerParams(
        needs_layout_passes=False, use_tc_tiling_on_sc=True),
    scratch_types=[
        pltpu.VMEM((_BW,), jnp.int32),
        pltpu.VMEM((_BW,), jnp.int32),
        pltpu.VMEM((2, 128), jnp.int32),
        pltpu.VMEM((2, 128), jnp.int32),
        pltpu.VMEM((2 * _C, 128), jnp.float32),
        pltpu.VMEM((2 * _C, 128), jnp.float32),
        pltpu.VMEM((16, 128), jnp.float32),
        pltpu.VMEM((_BW,), jnp.float32),
        pltpu.VMEM((8, 16), jnp.float32),
        pltpu.SemaphoreType.DMA,
    ],
)


_RBLK = 16000


def _tc_relayout_body(ueT_ref, ieT_ref, ue2_ref, ie2_ref):
    xcat = jnp.concatenate([ueT_ref[...], ieT_ref[...]], axis=0)
    xt = xcat.T
    for col0, dst in ((0, ue2_ref), (64, ie2_ref)):
        x3 = xt[:, col0:col0 + 64].reshape(_RBLK // 2, 2, 64)
        dst[...] = jnp.concatenate([x3[:, 0, :], x3[:, 1, :]], axis=1)


_tc_relayout = pl.pallas_call(
    _tc_relayout_body,
    grid=(400000 // _RBLK,),
    in_specs=[
        pl.BlockSpec((64, _RBLK), lambda j: (0, j)),
        pl.BlockSpec((64, _RBLK), lambda j: (0, j)),
    ],
    out_specs=[
        pl.BlockSpec((_RBLK // 2, 128), lambda j: (j, 0)),
        pl.BlockSpec((_RBLK // 2, 128), lambda j: (j, 0)),
    ],
    out_shape=[
        jax.ShapeDtypeStruct((200000, 128), jnp.float32),
        jax.ShapeDtypeStruct((200000, 128), jnp.float32),
    ],
)


def kernel(u, i, userEmbed_weight, itemEmbed_weight, userProb_weight):
    ue2, ie2 = _tc_relayout(userEmbed_weight.T, itemEmbed_weight.T)
    prob2 = jnp.take(userProb_weight, u, axis=0).reshape(_BW, 128)
    out, pair = _sc_call(u, i, ue2, ie2, prob2)
    s = jnp.sum(pair, axis=(0, 2))[:6]
    facet_loss = jnp.sum(jnp.log(1.0 + jnp.exp(-0.1 * s)))
    return out, facet_loss.astype(jnp.float32)
